# baseline (speedup 1.0000x reference)
import functools

import jax
import jax.numpy as jnp
from jax import lax
from jax.experimental import pallas as pl
from jax.experimental.pallas import tpu as pltpu
import jax.experimental.pallas.tpu_sc as plsc

N = 10000
E = 320000
D = 128

NC = 2
NS = 16
NW = NC * NS
EH = E // 2
EPW2 = EH // NW
CH = 40
NCH2 = EPW2 // CH
NRG = 4
NGRP = NCH2 // NRG
NTL = NCH2 - NGRP * NRG
EPW = E // NW
CHS = 40
NCHS = EPW // CHS
NPAD = 10112
RPS = NPAD // NS

_mesh = plsc.VectorSubcoreMesh(core_axis_name="c", subcore_axis_name="s",
                               num_cores=NC, num_subcores=NS)


@functools.partial(
    pl.kernel,
    out_type=(jax.ShapeDtypeStruct((EH, D), jnp.float32),
              jax.ShapeDtypeStruct((EH, D), jnp.float32)),
    mesh=_mesh,
    scratch_types=[
        pltpu.VMEM((NCH2, CH), jnp.int32),
        pltpu.VMEM((NCH2, CH), jnp.int32),
        pltpu.VMEM((NRG, CH, D), jnp.float32),
        pltpu.VMEM((NRG, CH, D), jnp.float32),
        [pltpu.SemaphoreType.DMA] * NRG,
        pltpu.SemaphoreType.DMA,
    ],
)
def _sc_gather_h(x_hbm, dst_hbm, src_hbm, xi_hbm, xj_hbm,
                 idx_d, idx_s, bufd, bufs, sems, sem_w):
    sid = lax.axis_index("s")
    wid = sid * NC + lax.axis_index("c")
    base = wid * EPW2
    pltpu.sync_copy(dst_hbm.at[wid], idx_d)
    pltpu.sync_copy(src_hbm.at[wid], idx_s)

    def drain_w(r):
        pltpu.make_async_copy(bufd.at[r], xi_hbm.at[pl.ds(base, CH)],
                              sem_w).wait()
        pltpu.make_async_copy(bufs.at[r], xj_hbm.at[pl.ds(base, CH)],
                              sem_w).wait()

    def step(g, carry):
        k0 = g * NRG
        gops = []
        for r in range(NRG):
            k = k0 + r

            @pl.when(g > 0)
            def _(r=r):
                drain_w(r)

            g1 = pltpu.async_copy(x_hbm.at[idx_d.at[k]], bufd.at[r], sems[r])
            g2 = pltpu.async_copy(x_hbm.at[idx_s.at[k]], bufs.at[r], sems[r])
            gops.append((g1, g2))
        for r in range(NRG):
            k = k0 + r
            row = base + k * CH
            gops[r][0].wait()
            gops[r][1].wait()
            pltpu.async_copy(bufd.at[r], xi_hbm.at[pl.ds(row, CH)], sem_w)
            pltpu.async_copy(bufs.at[r], xj_hbm.at[pl.ds(row, CH)], sem_w)
        return carry

    lax.fori_loop(0, NGRP, step, 0)

    tops = []
    for r in range(NTL):
        k = NGRP * NRG + r
        drain_w(r)
        g1 = pltpu.async_copy(x_hbm.at[idx_d.at[k]], bufd.at[r], sems[r])
        g2 = pltpu.async_copy(x_hbm.at[idx_s.at[k]], bufs.at[r], sems[r])
        tops.append((g1, g2))
    for r in range(NTL):
        k = NGRP * NRG + r
        row = base + k * CH
        tops[r][0].wait()
        tops[r][1].wait()
        pltpu.async_copy(bufd.at[r], xi_hbm.at[pl.ds(row, CH)], sem_w)
        pltpu.async_copy(bufs.at[r], xj_hbm.at[pl.ds(row, CH)], sem_w)
    for r in range(NRG):
        drain_w(r)


BE = 8000


def _dense_body(xi_ref, xj_ref, ea_ref, wa_ref, wb_ref, wc_ref, b_ref, out_ref):
    bf = jnp.bfloat16
    acc = jnp.dot(xi_ref[...].astype(bf), wa_ref[...],
                  preferred_element_type=jnp.float32)
    acc += jnp.dot(xj_ref[...].astype(bf), wb_ref[...],
                   preferred_element_type=jnp.float32)
    acc += jnp.dot(ea_ref[...].astype(bf), wc_ref[...],
                   preferred_element_type=jnp.float32)
    acc += b_ref[...]
    lf = acc[:, :D]
    ls = acc[:, D:]
    gate = 1.0 / (1.0 + jnp.exp(-lf))
    core = jnp.maximum(ls, 0.0) + jnp.log1p(jnp.exp(-jnp.abs(ls)))
    out_ref[...] = gate * core


def _dense_half(xi, xj, ea, wa, wb, wc, b, half):
    off = half * (EH // BE)
    return pl.pallas_call(
        _dense_body,
        grid=(EH // BE,),
        in_specs=[
            pl.BlockSpec((BE, D), lambda i: (i, 0)),
            pl.BlockSpec((BE, D), lambda i: (i, 0)),
            pl.BlockSpec((BE, D), lambda i: (i + off, 0)),
            pl.BlockSpec((D, 2 * D), lambda i: (0, 0)),
            pl.BlockSpec((D, 2 * D), lambda i: (0, 0)),
            pl.BlockSpec((D, 2 * D), lambda i: (0, 0)),
            pl.BlockSpec((1, 2 * D), lambda i: (0, 0)),
        ],
        out_specs=pl.BlockSpec((BE, D), lambda i: (i, 0)),
        out_shape=jax.ShapeDtypeStruct((EH, D), jnp.float32),
    )(xi, xj, ea, wa, wb, wc, b)


@functools.partial(
    pl.kernel,
    out_type=jax.ShapeDtypeStruct((NC, NPAD, D), jnp.float32),
    mesh=_mesh,
    scratch_types=[
        pltpu.VMEM((NCHS, CHS), jnp.int32),
        pltpu.VMEM((CHS, D), jnp.float32),
        pltpu.VMEM((CHS, D), jnp.float32),
        pltpu.VMEM_SHARED((NPAD, D), jnp.float32),
        pltpu.SemaphoreType.DMA,
        pltpu.SemaphoreType.DMA,
    ],
)
def _sc_scatter(msg0_hbm, msg1_hbm, dst_hbm, zero_hbm, p_hbm,
                idx, rowsa, rowsb, accum, sem_a, sem_b):
    cid = lax.axis_index("c")
    sid = lax.axis_index("s")
    wid = sid * NC + cid
    pltpu.sync_copy(zero_hbm.at[pl.ds(sid * RPS, RPS)],
                    accum.at[pl.ds(sid * RPS, RPS)])
    pltpu.sync_copy(dst_hbm.at[wid], idx)
    plsc.subcore_barrier()

    def run(msg_hbm, lbase):
        def step(j, carry):
            a = 2 * j
            b = a + 1
            ra = pltpu.async_copy(msg_hbm.at[pl.ds(lbase + a * CHS, CHS)],
                                  rowsa, sem_a)
            rb = pltpu.async_copy(msg_hbm.at[pl.ds(lbase + b * CHS, CHS)],
                                  rowsb, sem_b)
            ra.wait()
            pltpu.sync_copy(rowsa, accum.at[idx.at[a]], add=True)
            rb.wait()
            pltpu.sync_copy(rowsb, accum.at[idx.at[b]], add=True)
            return carry

        lax.fori_loop(0, NCHS // 2, step, 0)

    @pl.when(wid < NW // 2)
    def _():
        run(msg0_hbm, wid * EPW)

    @pl.when(wid >= NW // 2)
    def _():
        run(msg1_hbm, wid * EPW - EH)

    plsc.subcore_barrier()
    pltpu.sync_copy(accum.at[pl.ds(sid * RPS, RPS)],
                    p_hbm.at[cid, pl.ds(sid * RPS, RPS)])


BN = 1000


def _epi_body(p0_ref, p1_ref, x_ref, out_ref):
    out_ref[...] = p0_ref[0] + p1_ref[0] + x_ref[...]


def _epilogue(p, x):
    return pl.pallas_call(
        _epi_body,
        grid=(N // BN,),
        in_specs=[
            pl.BlockSpec((1, BN, D), lambda i: (0, i, 0)),
            pl.BlockSpec((1, BN, D), lambda i: (1, i, 0)),
            pl.BlockSpec((BN, D), lambda i: (i, 0)),
        ],
        out_specs=pl.BlockSpec((BN, D), lambda i: (i, 0)),
        out_shape=jax.ShapeDtypeStruct((N, D), jnp.float32),
    )(p, p, x)


def kernel(x, edge_index, edge_attr, W_f, b_f, W_s, b_s):
    ei = edge_index.astype(jnp.int32)
    d0 = ei[1, :EH].reshape(NW, NCH2, CH)
    s0 = ei[0, :EH].reshape(NW, NCH2, CH)
    d1 = ei[1, EH:].reshape(NW, NCH2, CH)
    s1 = ei[0, EH:].reshape(NW, NCH2, CH)
    dst3s = ei[1].reshape(NW, NCHS, CHS)

    xi0, xj0 = _sc_gather_h(x, d0, s0)
    xi1, xj1 = _sc_gather_h(x, d1, s1)

    wa = jnp.concatenate([W_f[:D], W_s[:D]], axis=1).astype(jnp.bfloat16)
    wb = jnp.concatenate([W_f[D:2 * D], W_s[D:2 * D]], axis=1).astype(jnp.bfloat16)
    wc = jnp.concatenate([W_f[2 * D:], W_s[2 * D:]], axis=1).astype(jnp.bfloat16)
    b = jnp.concatenate([b_f, b_s]).reshape(1, 2 * D)
    msg0 = _dense_half(xi0, xj0, edge_attr, wa, wb, wc, b, 0)
    msg1 = _dense_half(xi1, xj1, edge_attr, wa, wb, wc, b, 1)

    zero = jnp.zeros((NPAD, D), jnp.float32)
    p = _sc_scatter(msg0, msg1, dst3s, zero)
    return _epilogue(p, x)


# speedup vs baseline: 1.0536x; 1.0536x over previous
import functools

import jax
import jax.numpy as jnp
from jax import lax
from jax.experimental import pallas as pl
from jax.experimental.pallas import tpu as pltpu
import jax.experimental.pallas.tpu_sc as plsc

N = 10000
E = 320000
D = 128

NC = 2
NS = 16
NW = NC * NS
EH = E // 2
EPW2 = EH // NW
CH = 40
NCH2 = EPW2 // CH
NRG = 3
NGRP = NCH2 // NRG
NTL = NCH2 - NGRP * NRG
EPW = E // NW
CHS = 80
NCHS = EPW // CHS
NPAD = 10112
RPS = NPAD // NS

_mesh = plsc.VectorSubcoreMesh(core_axis_name="c", subcore_axis_name="s",
                               num_cores=NC, num_subcores=NS)


@functools.partial(
    pl.kernel,
    out_type=(jax.ShapeDtypeStruct((EH, D), jnp.float32),
              jax.ShapeDtypeStruct((EH, D), jnp.float32)),
    mesh=_mesh,
    scratch_types=[
        pltpu.VMEM((NCH2, CH), jnp.int32),
        pltpu.VMEM((NCH2, CH), jnp.int32),
        pltpu.VMEM((NRG, CH, D), jnp.float32),
        pltpu.VMEM((NRG, CH, D), jnp.float32),
        [pltpu.SemaphoreType.DMA] * NRG,
        pltpu.SemaphoreType.DMA,
    ],
)
def _sc_gather_h(x_hbm, dst_hbm, src_hbm, xi_hbm, xj_hbm,
                 idx_d, idx_s, bufd, bufs, sems, sem_w):
    sid = lax.axis_index("s")
    wid = sid * NC + lax.axis_index("c")
    base = wid * EPW2
    pltpu.sync_copy(dst_hbm.at[wid], idx_d)
    pltpu.sync_copy(src_hbm.at[wid], idx_s)

    def drain_w(r):
        pltpu.make_async_copy(bufd.at[r], xi_hbm.at[pl.ds(base, CH)],
                              sem_w).wait()
        pltpu.make_async_copy(bufs.at[r], xj_hbm.at[pl.ds(base, CH)],
                              sem_w).wait()

    def step(g, carry):
        k0 = g * NRG
        gops = []
        for r in range(NRG):
            k = k0 + r

            @pl.when(g > 0)
            def _(r=r):
                drain_w(r)

            g1 = pltpu.async_copy(x_hbm.at[idx_d.at[k]], bufd.at[r], sems[r])
            g2 = pltpu.async_copy(x_hbm.at[idx_s.at[k]], bufs.at[r], sems[r])
            gops.append((g1, g2))
        for r in range(NRG):
            k = k0 + r
            row = base + k * CH
            gops[r][0].wait()
            gops[r][1].wait()
            pltpu.async_copy(bufd.at[r], xi_hbm.at[pl.ds(row, CH)], sem_w)
            pltpu.async_copy(bufs.at[r], xj_hbm.at[pl.ds(row, CH)], sem_w)
        return carry

    lax.fori_loop(0, NGRP, step, 0)

    tops = []
    for r in range(NTL):
        k = NGRP * NRG + r
        drain_w(r)
        g1 = pltpu.async_copy(x_hbm.at[idx_d.at[k]], bufd.at[r], sems[r])
        g2 = pltpu.async_copy(x_hbm.at[idx_s.at[k]], bufs.at[r], sems[r])
        tops.append((g1, g2))
    for r in range(NTL):
        k = NGRP * NRG + r
        row = base + k * CH
        tops[r][0].wait()
        tops[r][1].wait()
        pltpu.async_copy(bufd.at[r], xi_hbm.at[pl.ds(row, CH)], sem_w)
        pltpu.async_copy(bufs.at[r], xj_hbm.at[pl.ds(row, CH)], sem_w)
    for r in range(NRG):
        drain_w(r)


BE = 8000


def _dense_body(xi_ref, xj_ref, ea_ref, wa_ref, wb_ref, wc_ref, b_ref, out_ref):
    bf = jnp.bfloat16
    acc = jnp.dot(xi_ref[...].astype(bf), wa_ref[...],
                  preferred_element_type=jnp.float32)
    acc += jnp.dot(xj_ref[...].astype(bf), wb_ref[...],
                   preferred_element_type=jnp.float32)
    acc += jnp.dot(ea_ref[...].astype(bf), wc_ref[...],
                   preferred_element_type=jnp.float32)
    acc += b_ref[...]
    lf = acc[:, :D]
    ls = acc[:, D:]
    gate = 1.0 / (1.0 + jnp.exp(-lf))
    core = jnp.maximum(ls, 0.0) + jnp.log1p(jnp.exp(-jnp.abs(ls)))
    out_ref[...] = gate * core


def _dense_half(xi, xj, ea, wa, wb, wc, b, half):
    off = half * (EH // BE)
    return pl.pallas_call(
        _dense_body,
        grid=(EH // BE,),
        in_specs=[
            pl.BlockSpec((BE, D), lambda i: (i, 0)),
            pl.BlockSpec((BE, D), lambda i: (i, 0)),
            pl.BlockSpec((BE, D), lambda i: (i + off, 0)),
            pl.BlockSpec((D, 2 * D), lambda i: (0, 0)),
            pl.BlockSpec((D, 2 * D), lambda i: (0, 0)),
            pl.BlockSpec((D, 2 * D), lambda i: (0, 0)),
            pl.BlockSpec((1, 2 * D), lambda i: (0, 0)),
        ],
        out_specs=pl.BlockSpec((BE, D), lambda i: (i, 0)),
        out_shape=jax.ShapeDtypeStruct((EH, D), jnp.float32),
    )(xi, xj, ea, wa, wb, wc, b)


@functools.partial(
    pl.kernel,
    out_type=jax.ShapeDtypeStruct((NC, NPAD, D), jnp.float32),
    mesh=_mesh,
    scratch_types=[
        pltpu.VMEM((NCHS, CHS), jnp.int32),
        pltpu.VMEM((CHS, D), jnp.float32),
        pltpu.VMEM((CHS, D), jnp.float32),
        pltpu.VMEM_SHARED((NPAD, D), jnp.float32),
        pltpu.SemaphoreType.DMA,
        pltpu.SemaphoreType.DMA,
    ],
)
def _sc_scatter(msg0_hbm, msg1_hbm, dst_hbm, zero_hbm, p_hbm,
                idx, rowsa, rowsb, accum, sem_a, sem_b):
    cid = lax.axis_index("c")
    sid = lax.axis_index("s")
    wid = sid * NC + cid
    pltpu.sync_copy(zero_hbm.at[pl.ds(sid * RPS, RPS)],
                    accum.at[pl.ds(sid * RPS, RPS)])
    pltpu.sync_copy(dst_hbm.at[wid], idx)
    plsc.subcore_barrier()

    def run(msg_hbm, lbase):
        def step(j, carry):
            a = 2 * j
            b = a + 1
            ra = pltpu.async_copy(msg_hbm.at[pl.ds(lbase + a * CHS, CHS)],
                                  rowsa, sem_a)
            rb = pltpu.async_copy(msg_hbm.at[pl.ds(lbase + b * CHS, CHS)],
                                  rowsb, sem_b)
            ra.wait()
            pltpu.sync_copy(rowsa, accum.at[idx.at[a]], add=True)
            rb.wait()
            pltpu.sync_copy(rowsb, accum.at[idx.at[b]], add=True)
            return carry

        lax.fori_loop(0, NCHS // 2, step, 0)
        rt = pltpu.async_copy(msg_hbm.at[pl.ds(lbase + (NCHS - 1) * CHS, CHS)],
                              rowsa, sem_a)
        rt.wait()
        pltpu.sync_copy(rowsa, accum.at[idx.at[NCHS - 1]], add=True)

    @pl.when(wid < NW // 2)
    def _():
        run(msg0_hbm, wid * EPW)

    @pl.when(wid >= NW // 2)
    def _():
        run(msg1_hbm, wid * EPW - EH)

    plsc.subcore_barrier()
    pltpu.sync_copy(accum.at[pl.ds(sid * RPS, RPS)],
                    p_hbm.at[cid, pl.ds(sid * RPS, RPS)])


BN = 1000


def _epi_body(p0_ref, p1_ref, x_ref, out_ref):
    out_ref[...] = p0_ref[0] + p1_ref[0] + x_ref[...]


def _epilogue(p, x):
    return pl.pallas_call(
        _epi_body,
        grid=(N // BN,),
        in_specs=[
            pl.BlockSpec((1, BN, D), lambda i: (0, i, 0)),
            pl.BlockSpec((1, BN, D), lambda i: (1, i, 0)),
            pl.BlockSpec((BN, D), lambda i: (i, 0)),
        ],
        out_specs=pl.BlockSpec((BN, D), lambda i: (i, 0)),
        out_shape=jax.ShapeDtypeStruct((N, D), jnp.float32),
    )(p, p, x)


def kernel(x, edge_index, edge_attr, W_f, b_f, W_s, b_s):
    ei = edge_index.astype(jnp.int32)
    d0 = ei[1, :EH].reshape(NW, NCH2, CH)
    s0 = ei[0, :EH].reshape(NW, NCH2, CH)
    d1 = ei[1, EH:].reshape(NW, NCH2, CH)
    s1 = ei[0, EH:].reshape(NW, NCH2, CH)
    dst3s = ei[1].reshape(NW, NCHS, CHS)

    xi0, xj0 = _sc_gather_h(x, d0, s0)
    xi1, xj1 = _sc_gather_h(x, d1, s1)

    wa = jnp.concatenate([W_f[:D], W_s[:D]], axis=1).astype(jnp.bfloat16)
    wb = jnp.concatenate([W_f[D:2 * D], W_s[D:2 * D]], axis=1).astype(jnp.bfloat16)
    wc = jnp.concatenate([W_f[2 * D:], W_s[2 * D:]], axis=1).astype(jnp.bfloat16)
    b = jnp.concatenate([b_f, b_s]).reshape(1, 2 * D)
    msg0 = _dense_half(xi0, xj0, edge_attr, wa, wb, wc, b, 0)
    msg1 = _dense_half(xi1, xj1, edge_attr, wa, wb, wc, b, 1)

    zero = jnp.zeros((NPAD, D), jnp.float32)
    p = _sc_scatter(msg0, msg1, dst3s, zero)
    return _epilogue(p, x)
